# baseline (speedup 1.0000x reference)
import functools

import jax
import jax.numpy as jnp
from jax import lax
from jax.experimental import pallas as pl
from jax.experimental.pallas import tpu as pltpu
from jax.experimental.pallas import tpu_sc as plsc

_NUM_EMBEDDINGS = 8192
_DIM = 256
_COMMIT = 0.25

_ROWS = 32768
_RT = 256
_NT = 1024
_BIG = 2 ** 30


_W0, _W1 = 2736, 5472


def _argmin_body(x_ref, a_ref, b_ref, e_ref, out_ref):
    x = x_ref[...]
    a_col = a_ref[:, 0:1]
    run_min = [jnp.full((_RT, 1), jnp.inf, jnp.float32) for _ in range(3)]
    run_idx = [jnp.full((_RT, 1), _BIG, jnp.int32) for _ in range(3)]
    iota0 = lax.broadcasted_iota(jnp.int32, (_RT, _NT), 1)
    inf = jnp.float32(jnp.inf)
    for n in range(_NUM_EMBEDDINGS // _NT):
        lo = n * _NT
        e_blk = e_ref[lo:lo + _NT, :]
        s = lax.dot_general(x, e_blk, (((1,), (1,)), ((), ())),
                            preferred_element_type=jnp.float32)
        d = (a_col + b_ref[0:1, lo:lo + _NT]) - 2.0 * s
        gidx = iota0 + lo
        for w in range(3):
            wlo = (0, _W0, _W1)[w]
            whi = (_W0, _W1, _NUM_EMBEDDINGS)[w]
            if whi <= lo or wlo >= lo + _NT:
                continue
            if wlo <= lo and whi >= lo + _NT:
                dm = d
            else:
                mask = (gidx >= wlo) & (gidx < whi)
                dm = jnp.where(mask, d, inf)
            cmin = jnp.min(dm, axis=1, keepdims=True)
            cidx = jnp.min(jnp.where(dm == cmin, gidx, _BIG),
                           axis=1, keepdims=True)
            better = cmin < run_min[w]
            run_idx[w] = jnp.where(better, cidx, run_idx[w])
            run_min[w] = jnp.where(better, cmin, run_min[w])
    acc_v = run_min[0].astype(jnp.bfloat16).astype(jnp.float32)
    acc_i = run_idx[0]
    for w in (1, 2):
        upd = run_min[w] < acc_v
        acc_i = jnp.where(upd, run_idx[w], acc_i)
        if w < 2:
            acc_v = jnp.where(
                upd, run_min[w].astype(jnp.bfloat16).astype(jnp.float32),
                acc_v)
    out_ref[...] = jnp.broadcast_to(acc_i, (_RT, 128))


def _combine_body(q_ref, x_ref, out_ref, loss_ref):
    q = q_ref[...]
    x = x_ref[...]
    diff = q - x
    qd = x + diff
    out_ref[...] = jnp.transpose(qd)[None, :, :]
    @pl.when(pl.program_id(0) == 0)
    def _():
        loss_ref[0, 0] = jnp.float32(0.0)
    loss_ref[0, 0] += jnp.sum(diff * diff)


def _sc_gather(embedding, idx2d):
    mesh = plsc.VectorSubcoreMesh(core_axis_name="c", subcore_axis_name="s")

    @functools.partial(
        pl.kernel,
        mesh=mesh,
        out_type=jax.ShapeDtypeStruct((_ROWS, _DIM), jnp.float32),
        scratch_types=[
            pltpu.VMEM((8, 128), jnp.int32),
            pltpu.VMEM((128, _DIM), jnp.float32),
            pltpu.VMEM((128, _DIM), jnp.float32),
            pltpu.SemaphoreType.DMA,
            pltpu.SemaphoreType.DMA,
        ],
    )
    def k(table_hbm, idx_hbm, out_hbm, idx_v, buf0, buf1, sem0, sem1):
        wid = lax.axis_index("s") * 2 + lax.axis_index("c")
        base = wid * 1024
        pltpu.sync_copy(idx_hbm.at[pl.ds(wid * 8, 8)], idx_v)
        bufs = (buf0, buf1)
        sems = (sem0, sem1)
        copies = [None] * 8
        copies[0] = pltpu.async_copy(table_hbm.at[idx_v.at[0]], bufs[0],
                                     sems[0])
        for c in range(8):
            copies[c].wait()
            if c + 1 < 8:
                copies[c + 1] = pltpu.async_copy(
                    table_hbm.at[idx_v.at[c + 1]], bufs[(c + 1) % 2],
                    sems[(c + 1) % 2])
            pltpu.sync_copy(bufs[c % 2],
                            out_hbm.at[pl.ds(base + c * 128, 128)])

    return k(embedding, idx2d)


def kernel(inputs, embedding):
    bsz, ch, dd, hh, ww = inputs.shape
    x5 = jnp.transpose(inputs, (0, 2, 3, 4, 1))
    flat = x5.reshape(-1, _DIM)
    a = jnp.sum(flat ** 2, axis=1, keepdims=True)
    a_b = jnp.broadcast_to(a, (_ROWS, 128))
    b = jnp.sum(embedding ** 2, axis=1).reshape(1, _NUM_EMBEDDINGS)

    grid_a = _ROWS // _RT
    idx_b = pl.pallas_call(
        _argmin_body,
        grid=(grid_a,),
        in_specs=[
            pl.BlockSpec((_RT, _DIM), lambda r: (r, 0)),
            pl.BlockSpec((_RT, 128), lambda r: (r, 0)),
            pl.BlockSpec((1, _NUM_EMBEDDINGS), lambda r: (0, 0)),
            pl.BlockSpec((_NUM_EMBEDDINGS, _DIM), lambda r: (0, 0)),
        ],
        out_specs=pl.BlockSpec((_RT, 128), lambda r: (r, 0)),
        out_shape=jax.ShapeDtypeStruct((_ROWS, 128), jnp.int32),
        compiler_params=pltpu.CompilerParams(
            dimension_semantics=("arbitrary",)),
    )(flat, a_b, b, embedding)
    idx = idx_b[:, 0]

    q = _sc_gather(embedding, idx.reshape(256, 128))

    grid_c = _ROWS // 512
    out_t, loss_sum = pl.pallas_call(
        _combine_body,
        grid=(grid_c,),
        in_specs=[
            pl.BlockSpec((512, _DIM), lambda r: (r, 0)),
            pl.BlockSpec((512, _DIM), lambda r: (r, 0)),
        ],
        out_specs=[
            pl.BlockSpec((1, _DIM, 512), lambda r: (r // 16, 0, r % 16)),
            pl.BlockSpec(memory_space=pltpu.SMEM),
        ],
        out_shape=[
            jax.ShapeDtypeStruct((bsz, _DIM, dd * hh * ww), jnp.float32),
            jax.ShapeDtypeStruct((1, 1), jnp.float32),
        ],
        compiler_params=pltpu.CompilerParams(
            dimension_semantics=("arbitrary",)),
    )(q, flat)

    qd = out_t.reshape(bsz, ch, dd, hh, ww)
    t = loss_sum[0, 0] / jnp.float32(_ROWS * _DIM)
    vq_loss = t + _COMMIT * t
    return (qd, vq_loss, idx)


# speedup vs baseline: 1.0914x; 1.0914x over previous
import functools

import jax
import jax.numpy as jnp
from jax import lax
from jax.experimental import pallas as pl
from jax.experimental.pallas import tpu as pltpu
from jax.experimental.pallas import tpu_sc as plsc

_NUM_EMBEDDINGS = 8192
_DIM = 256
_COMMIT = 0.25

_ROWS = 32768
_RT = 256
_NT = 1024
_BIG = 2 ** 30
_W0, _W1 = 2736, 5472


def _argmin_body(xt_ref, b_ref, e_ref, out_ref):
    xt = xt_ref[0]
    a_row = jnp.sum(xt * xt, axis=0, keepdims=True)
    run_min = [jnp.full((1, _RT), jnp.inf, jnp.float32) for _ in range(3)]
    run_idx = [jnp.full((1, _RT), _BIG, jnp.int32) for _ in range(3)]
    iota0 = lax.broadcasted_iota(jnp.int32, (_NT, _RT), 0)
    inf = jnp.float32(jnp.inf)
    for n in range(_NUM_EMBEDDINGS // _NT):
        lo = n * _NT
        e_blk = e_ref[lo:lo + _NT, :]
        s = lax.dot_general(e_blk, xt, (((1,), (0,)), ((), ())),
                            preferred_element_type=jnp.float32)
        d = (a_row + b_ref[lo:lo + _NT, 0:1]) - 2.0 * s
        gidx = iota0 + lo
        for w in range(3):
            wlo = (0, _W0, _W1)[w]
            whi = (_W0, _W1, _NUM_EMBEDDINGS)[w]
            if whi <= lo or wlo >= lo + _NT:
                continue
            if wlo <= lo and whi >= lo + _NT:
                dm = d
            else:
                mask = (gidx >= wlo) & (gidx < whi)
                dm = jnp.where(mask, d, inf)
            cmin = jnp.min(dm, axis=0, keepdims=True)
            cidx = jnp.min(jnp.where(dm == cmin, gidx, _BIG),
                           axis=0, keepdims=True)
            better = cmin < run_min[w]
            run_idx[w] = jnp.where(better, cidx, run_idx[w])
            run_min[w] = jnp.where(better, cmin, run_min[w])
    acc_v = run_min[0].astype(jnp.bfloat16).astype(jnp.float32)
    acc_i = run_idx[0]
    for w in (1, 2):
        upd = run_min[w] < acc_v
        acc_i = jnp.where(upd, run_idx[w], acc_i)
        if w < 2:
            acc_v = jnp.where(
                upd, run_min[w].astype(jnp.bfloat16).astype(jnp.float32),
                acc_v)
    out_ref[...] = acc_i[None]


def _combine_body(q_ref, xt_ref, out_ref, loss_ref):
    qt = jnp.transpose(q_ref[...])
    xt = xt_ref[0]
    diff = qt - xt
    out_ref[...] = (xt + diff)[None]
    @pl.when(pl.program_id(0) == 0)
    def _():
        loss_ref[0, 0] = jnp.float32(0.0)
    loss_ref[0, 0] += jnp.sum(diff * diff)


def _sc_gather(embedding, idx2d):
    mesh = plsc.VectorSubcoreMesh(core_axis_name="c", subcore_axis_name="s")

    @functools.partial(
        pl.kernel,
        mesh=mesh,
        out_type=jax.ShapeDtypeStruct((_ROWS, _DIM), jnp.float32),
        scratch_types=[
            pltpu.VMEM((8, 128), jnp.int32),
            pltpu.VMEM((128, _DIM), jnp.float32),
            pltpu.VMEM((128, _DIM), jnp.float32),
            pltpu.SemaphoreType.DMA,
            pltpu.SemaphoreType.DMA,
        ],
    )
    def k(table_hbm, idx_hbm, out_hbm, idx_v, buf0, buf1, sem0, sem1):
        wid = lax.axis_index("s") * 2 + lax.axis_index("c")
        base = wid * 1024
        pltpu.sync_copy(idx_hbm.at[pl.ds(wid * 8, 8)], idx_v)
        bufs = (buf0, buf1)
        sems = (sem0, sem1)
        copies = [None] * 8
        copies[0] = pltpu.async_copy(table_hbm.at[idx_v.at[0]], bufs[0],
                                     sems[0])
        for c in range(8):
            copies[c].wait()
            if c + 1 < 8:
                copies[c + 1] = pltpu.async_copy(
                    table_hbm.at[idx_v.at[c + 1]], bufs[(c + 1) % 2],
                    sems[(c + 1) % 2])
            pltpu.sync_copy(bufs[c % 2],
                            out_hbm.at[pl.ds(base + c * 128, 128)])

    return k(embedding, idx2d)


def kernel(inputs, embedding):
    bsz, ch, dd, hh, ww = inputs.shape
    sp = dd * hh * ww
    x3 = inputs.reshape(bsz, ch, sp)
    b = jnp.sum(embedding ** 2, axis=1)
    b_bc = jnp.broadcast_to(b[:, None], (_NUM_EMBEDDINGS, 128))

    grid_a = _ROWS // _RT
    tiles_pb = sp // _RT
    idx_t = pl.pallas_call(
        _argmin_body,
        grid=(grid_a,),
        in_specs=[
            pl.BlockSpec((1, _DIM, _RT), lambda r: (r // 32, 0, r % 32)),
            pl.BlockSpec((_NUM_EMBEDDINGS, 128), lambda r: (0, 0)),
            pl.BlockSpec((_NUM_EMBEDDINGS, _DIM), lambda r: (0, 0)),
        ],
        out_specs=pl.BlockSpec((1, 1, _RT), lambda r: (r, 0, 0)),
        out_shape=jax.ShapeDtypeStruct((grid_a, 1, _RT), jnp.int32),
        compiler_params=pltpu.CompilerParams(
            dimension_semantics=("arbitrary",)),
    )(x3, b_bc, embedding)
    idx = idx_t.reshape(_ROWS)

    q = _sc_gather(embedding, idx.reshape(256, 128))

    grid_c = _ROWS // 512
    out3, loss_sum = pl.pallas_call(
        _combine_body,
        grid=(grid_c,),
        in_specs=[
            pl.BlockSpec((512, _DIM), lambda r: (r, 0)),
            pl.BlockSpec((1, _DIM, 512), lambda r: (r // 16, 0, r % 16)),
        ],
        out_specs=[
            pl.BlockSpec((1, _DIM, 512), lambda r: (r // 16, 0, r % 16)),
            pl.BlockSpec(memory_space=pltpu.SMEM),
        ],
        out_shape=[
            jax.ShapeDtypeStruct((bsz, _DIM, sp), jnp.float32),
            jax.ShapeDtypeStruct((1, 1), jnp.float32),
        ],
        compiler_params=pltpu.CompilerParams(
            dimension_semantics=("arbitrary",)),
    )(q, x3)

    qd = out3.reshape(bsz, ch, dd, hh, ww)
    t = loss_sum[0, 0] / jnp.float32(_ROWS * _DIM)
    vq_loss = t + _COMMIT * t
    return (qd, vq_loss, idx)
